# baseline (speedup 1.0000x reference)
import math

import jax
import jax.numpy as jnp
import numpy as np
from jax.experimental import pallas as pl
from jax.experimental.pallas import tpu as pltpu

D_INP = 36
D_MODEL = 144
D_PE = 36
NHEAD = 4
MAX_LEN = 215
T_PAD = 216
D_TRANS = 180
HEAD_DIM = 45
NHID = 128
D_FINAL = 108
N_CLASSES = 2
BB = 8
NG = 40

_TS = np.asarray(MAX_LEN ** np.linspace(0.0, 1.0, D_PE // 2), np.float32)
_SKIP_DIST = False


def _dot(a, b):
    return jax.lax.dot_general(a, b, (((1,), (0,)), ((), ())),
                               preferred_element_type=jnp.float32)


def _dot_t(a, b):
    return jax.lax.dot_general(a, b, (((1,), (1,)), ((), ())),
                               preferred_element_type=jnp.float32)


def _ln(x, g, b):
    mu = jnp.mean(x, axis=1, keepdims=True)
    d = x - mu
    var = jnp.mean(d * d, axis=1, keepdims=True)
    return d / jnp.sqrt(var + 1e-5) * g + b


def _mega_kernel(xsrc_ref, t_ref, mk_ref, mq_ref, len_ref, adjt_ref, ts_ref,
                 Wenc_ref, benc_ref,
                 Wq_ref, bq_ref, Wk_ref, bk_ref, Wv_ref, bv_ref,
                 Wskip_ref, bskip_ref,
                 l0_Wqkv_ref, l0_bqkv_ref, l0_Wo_ref, l0_bo_ref,
                 l0_W1_ref, l0_b1_ref, l0_W2_ref, l0_b2_ref,
                 l0_g1_ref, l0_be1_ref, l0_g2_ref, l0_be2_ref,
                 l1_Wqkv_ref, l1_bqkv_ref, l1_Wo_ref, l1_bo_ref,
                 l1_W1_ref, l1_b1_ref, l1_W2_ref, l1_b2_ref,
                 l1_g1_ref, l1_be1_ref, l1_g2_ref, l1_be2_ref,
                 Wm1_ref, bm1_ref, Wm2_ref, bm2_ref,
                 out_ref, attn_ref):
    f32 = jnp.float32

    x = xsrc_ref[...]
    x = (_dot(x, Wenc_ref[...]) + benc_ref[...]) * 12.0

    xg = jnp.concatenate([x[j * T_PAD:j * T_PAD + NG] for j in range(BB)],
                         axis=0)
    q = _dot(xg, Wq_ref[...]) + bq_ref[...]
    k = _dot(xg, Wk_ref[...]) + bk_ref[...]
    v = _dot(xg, Wv_ref[...]) + bv_ref[...]
    cmask = jnp.where(
        jax.lax.broadcasted_iota(jnp.int32, (1, NG), 1) >= D_INP,
        f32(-1e9), f32(0.0))
    ag_rows = []
    for j in range(BB):
        qj = q[j * NG:(j + 1) * NG]
        kj = k[j * NG:(j + 1) * NG]
        vj = v[j * NG:(j + 1) * NG]
        s = _dot_t(qj, kj) * (1.0 / 12.0)
        s = s * adjt_ref[...] + cmask
        m = jnp.max(s, axis=1, keepdims=True)
        e = jnp.exp(s - m)
        attn = e / (jnp.sum(e, axis=1, keepdims=True) + 1e-16)
        attn_ref[j] = attn[:D_INP, :D_INP]
        agj = _dot(attn, vj)
        ag_rows.append(jnp.concatenate(
            [agj[:D_INP], jnp.zeros((T_PAD - D_INP, D_MODEL), f32)], axis=0))
    skip = _dot(x, Wskip_ref[...]) + bskip_ref[...]
    outs = skip + jnp.concatenate(ag_rows, axis=0)

    t = t_ref[...]
    scaled = t / ts_ref[...]
    pe = jnp.concatenate([jnp.sin(scaled), jnp.cos(scaled)], axis=1)

    y = jnp.concatenate([outs, pe], axis=1)

    negm = (mk_ref[...] - 1.0) * 1e9

    def enc_layer(y, Wqkv, bqkv, Wo, bo, W1, b1, W2, b2, g1, be1, g2, be2):
        qkv = _dot(y, Wqkv) + bqkv
        blocks = []
        for j in range(BB):
            qkvj = qkv[j * T_PAD:(j + 1) * T_PAD]
            nm = negm[j]
            heads = []
            for h in range(NHEAD):
                o = h * HEAD_DIM
                qh = qkvj[:, o:o + HEAD_DIM]
                kh = qkvj[:, D_TRANS + o:D_TRANS + o + HEAD_DIM]
                vh = qkvj[:, 2 * D_TRANS + o:2 * D_TRANS + o + HEAD_DIM]
                s = _dot_t(qh, kh) * (1.0 / math.sqrt(HEAD_DIM))
                s = s + nm
                m = jnp.max(s, axis=1, keepdims=True)
                e = jnp.exp(s - m)
                heads.append(_dot(e, vh) /
                             jnp.sum(e, axis=1, keepdims=True))
            blocks.append(jnp.concatenate(heads, axis=1))
        o = jnp.concatenate(blocks, axis=0)
        o = _dot(o, Wo) + bo
        y = _ln(y + o, g1, be1)
        ff = _dot(jax.nn.relu(_dot(y, W1) + b1), W2) + b2
        return _ln(y + ff, g2, be2)

    y = enc_layer(y, l0_Wqkv_ref[...], l0_bqkv_ref[...], l0_Wo_ref[...],
                  l0_bo_ref[...], l0_W1_ref[...], l0_b1_ref[...],
                  l0_W2_ref[...], l0_b2_ref[...], l0_g1_ref[...],
                  l0_be1_ref[...], l0_g2_ref[...], l0_be2_ref[...])
    y = enc_layer(y, l1_Wqkv_ref[...], l1_bqkv_ref[...], l1_Wo_ref[...],
                  l1_bo_ref[...], l1_W1_ref[...], l1_b1_ref[...],
                  l1_W2_ref[...], l1_b2_ref[...], l1_g1_ref[...],
                  l1_be1_ref[...], l1_g2_ref[...], l1_be2_ref[...])

    yv = y * mq_ref[...]
    agg = jnp.concatenate(
        [jnp.sum(yv[j * T_PAD:(j + 1) * T_PAD], axis=0, keepdims=True)
         for j in range(BB)], axis=0)
    lenv = len_ref[...][:, 0, :]
    agg = agg / (lenv + 1.0)
    feat = agg[:, :D_FINAL]
    h1 = jax.nn.relu(_dot(feat, Wm1_ref[...]) + bm1_ref[...])
    outm = _dot(h1, Wm2_ref[...]) + bm2_ref[...]
    out_ref[...] = outm[:, None, :]


def _dist_kernel(a_ref, o_ref):
    a = a_ref[...]
    nb = a.shape[0]
    rb = 8

    def body(i, acc):
        blk = a_ref[pl.ds(i * rb, rb), :]
        diff = a[None, :, :] - blk[:, None, :]
        d2 = jnp.sum(diff * diff, axis=2)
        d = jnp.sqrt(jnp.maximum(d2, 1e-24))
        return acc + jnp.sum(d)

    tot = jax.lax.fori_loop(0, nb // rb, body, jnp.float32(0.0))
    o_ref[0, 0] = tot / float(nb * nb)


def kernel(src, static, times, lengths, adj, W_enc, b_enc, W_emb, b_emb,
           Wq, bq, Wk, bk, Wv, bv, Wskip, bskip,
           l0_Wqkv, l0_bqkv, l0_Wo, l0_bo, l0_W1, l0_b1, l0_W2, l0_b2,
           l0_ln1_g, l0_ln1_b, l0_ln2_g, l0_ln2_b,
           l1_Wqkv, l1_bqkv, l1_Wo, l1_bo, l1_W1, l1_b1, l1_W2, l1_b2,
           l1_ln1_g, l1_ln1_b, l1_ln2_g, l1_ln2_b,
           Wm1, bm1, Wm2, bm2):
    T, B = src.shape[0], src.shape[1]
    f32 = jnp.float32

    def pad_t(a):
        a = jnp.pad(a, ((0, 0), (0, T_PAD - T), (0, 0)))
        return a.reshape(B * T_PAD, a.shape[-1])

    xsrc = pad_t(jnp.transpose(src[:, :, :D_INP], (1, 0, 2)))
    t_r = pad_t(jnp.transpose(times, (1, 0))[:, :, None])
    valid = (jnp.arange(T_PAD)[None, :] < lengths[:, None]).astype(f32)
    mk = valid[:, None, :]
    mq = valid.reshape(B * T_PAD, 1)
    lenf = lengths.astype(f32)[:, None, None]
    adj_sl = adj.at[jnp.diag_indices(D_INP)].set(1.0)
    adjt = jnp.transpose(adj_sl, (1, 0))
    adjt = jnp.pad(adjt, ((0, NG - D_INP), (0, NG - D_INP)),
                   constant_values=1.0)

    r2 = lambda a: a.reshape(1, -1)

    def bspec(shape):
        nd = len(shape)
        return pl.BlockSpec(shape, lambda b, _n=nd: (0,) * _n)

    in_arrays = [
        xsrc, t_r, mk, mq, lenf, adjt, jnp.asarray(_TS)[None, :],
        W_enc, r2(b_enc),
        Wq, r2(bq), Wk, r2(bk), Wv, r2(bv), Wskip, r2(bskip),
        l0_Wqkv, r2(l0_bqkv), l0_Wo, r2(l0_bo),
        l0_W1, r2(l0_b1), l0_W2, r2(l0_b2),
        r2(l0_ln1_g), r2(l0_ln1_b), r2(l0_ln2_g), r2(l0_ln2_b),
        l1_Wqkv, r2(l1_bqkv), l1_Wo, r2(l1_bo),
        l1_W1, r2(l1_b1), l1_W2, r2(l1_b2),
        r2(l1_ln1_g), r2(l1_ln1_b), r2(l1_ln2_g), r2(l1_ln2_b),
        Wm1, r2(bm1), Wm2, r2(bm2),
    ]
    in_specs = [
        pl.BlockSpec((BB * T_PAD, D_INP), lambda b: (b, 0)),
        pl.BlockSpec((BB * T_PAD, 1), lambda b: (b, 0)),
        pl.BlockSpec((BB, 1, T_PAD), lambda b: (b, 0, 0)),
        pl.BlockSpec((BB * T_PAD, 1), lambda b: (b, 0)),
        pl.BlockSpec((BB, 1, 1), lambda b: (b, 0, 0)),
    ] + [bspec(a.shape) for a in in_arrays[5:]]

    out, attn = pl.pallas_call(
        _mega_kernel,
        grid=(B // BB,),
        in_specs=in_specs,
        out_specs=[
            pl.BlockSpec((BB, 1, N_CLASSES), lambda b: (b, 0, 0)),
            pl.BlockSpec((BB, D_INP, D_INP), lambda b: (b, 0, 0)),
        ],
        out_shape=[
            jax.ShapeDtypeStruct((B, 1, N_CLASSES), f32),
            jax.ShapeDtypeStruct((B, D_INP, D_INP), f32),
        ],
    )(*in_arrays)

    a_flat = attn.reshape(B, D_INP * D_INP)
    if _SKIP_DIST:
        return out.reshape(B, N_CLASSES), a_flat.sum() * 0.0
    distance = pl.pallas_call(
        _dist_kernel,
        out_specs=pl.BlockSpec(memory_space=pltpu.SMEM),
        out_shape=jax.ShapeDtypeStruct((1, 1), f32),
    )(a_flat)

    return out.reshape(B, N_CLASSES), distance.reshape(())


# speedup vs baseline: 1.0124x; 1.0124x over previous
import math

import jax
import jax.numpy as jnp
import numpy as np
from jax.experimental import pallas as pl
from jax.experimental.pallas import tpu as pltpu

D_INP = 36
D_MODEL = 144
D_PE = 36
NHEAD = 4
MAX_LEN = 215
T_PAD = 216
D_TRANS = 180
HEAD_DIM = 45
NHID = 128
D_FINAL = 108
N_CLASSES = 2
BB = 4
NG = 40

_TS = np.asarray(MAX_LEN ** np.linspace(0.0, 1.0, D_PE // 2), np.float32)
_TS2 = np.concatenate([_TS, _TS])[None, :]
_PH = np.concatenate([np.zeros(D_PE // 2, np.float32),
                      np.full(D_PE // 2, np.float32(np.pi / 2))])[None, :]
_P = np.zeros((BB * T_PAD, BB * NG), np.float32)
for _j in range(BB):
    for _r in range(D_INP):
        _P[_j * T_PAD + _r, _j * NG + _r] = 1.0
_S = np.zeros((BB, BB * T_PAD), np.float32)
for _j in range(BB):
    _S[_j, _j * T_PAD:(_j + 1) * T_PAD] = 1.0


def _dot(a, b):
    return jax.lax.dot_general(a, b, (((1,), (0,)), ((), ())),
                               preferred_element_type=jnp.float32)


def _dot_t(a, b):
    return jax.lax.dot_general(a, b, (((1,), (1,)), ((), ())),
                               preferred_element_type=jnp.float32)


def _ln(x, g, b):
    mu = jnp.mean(x, axis=1, keepdims=True)
    d = x - mu
    var = jnp.mean(d * d, axis=1, keepdims=True)
    return d / jnp.sqrt(var + 1e-5) * g + b


def _mega_kernel(xsrc_ref, t_ref, mk_ref, mq_ref, len_ref, adjt_ref,
                 ts2_ref, ph_ref, p_ref, s_ref,
                 Wenc_ref, benc_ref,
                 Wq_ref, bq_ref, Wk_ref, bk_ref, Wv_ref, bv_ref,
                 Wskip_ref, bskip_ref,
                 l0_Wqkv_ref, l0_bqkv_ref,
                 l0_Wo0_ref, l0_Wo1_ref, l0_Wo2_ref, l0_Wo3_ref, l0_bo_ref,
                 l0_W1_ref, l0_b1_ref, l0_W2_ref, l0_b2_ref,
                 l0_g1_ref, l0_be1_ref, l0_g2_ref, l0_be2_ref,
                 l1_Wqkv_ref, l1_bqkv_ref,
                 l1_Wo0_ref, l1_Wo1_ref, l1_Wo2_ref, l1_Wo3_ref, l1_bo_ref,
                 l1_W1_ref, l1_b1_ref, l1_W2_ref, l1_b2_ref,
                 l1_g1_ref, l1_be1_ref, l1_g2_ref, l1_be2_ref,
                 Wm1_ref, bm1_ref, Wm2_ref, bm2_ref,
                 out_ref, attn_ref):
    f32 = jnp.float32

    x = xsrc_ref[...]
    x = (_dot(x, Wenc_ref[...]) + benc_ref[...]) * 12.0

    xg = jnp.concatenate([x[j * T_PAD:j * T_PAD + NG] for j in range(BB)],
                         axis=0)
    q = _dot(xg, Wq_ref[...]) + bq_ref[...]
    k = _dot(xg, Wk_ref[...]) + bk_ref[...]
    v = _dot(xg, Wv_ref[...]) + bv_ref[...]
    cmask = jnp.where(
        jax.lax.broadcasted_iota(jnp.int32, (1, NG), 1) >= D_INP,
        f32(-1e9), f32(0.0))
    rmask = jnp.where(
        jax.lax.broadcasted_iota(jnp.int32, (NG, 1), 0) >= D_INP,
        f32(0.0), f32(1.0))
    ag_blocks = []
    for j in range(BB):
        qj = q[j * NG:(j + 1) * NG]
        kj = k[j * NG:(j + 1) * NG]
        vj = v[j * NG:(j + 1) * NG]
        s = _dot_t(qj, kj) * (1.0 / 12.0)
        s = s * adjt_ref[...] + cmask
        m = jnp.max(s, axis=1, keepdims=True)
        e = jnp.exp(s - m)
        attn = e / (jnp.sum(e, axis=1, keepdims=True) + 1e-16)
        attn_ref[j] = attn[:D_INP, :D_INP]
        ag_blocks.append(_dot(attn * rmask, vj))
    ag_all = jnp.concatenate(ag_blocks, axis=0)
    skip = _dot(x, Wskip_ref[...]) + bskip_ref[...]
    outs = skip + _dot(p_ref[...], ag_all)

    t = t_ref[...]
    pe = jnp.sin(t / ts2_ref[...] + ph_ref[...])

    y = jnp.concatenate([outs, pe], axis=1)

    negm = (mk_ref[...] - 1.0) * 1e9

    def enc_layer(y, Wqkv, bqkv, Wo_h, bo, W1, b1, W2, b2, g1, be1, g2, be2):
        qkv = _dot(y, Wqkv) + bqkv
        blocks = []
        for j in range(BB):
            qkvj = qkv[j * T_PAD:(j + 1) * T_PAD]
            nm = negm[j]
            oj = None
            for h in range(NHEAD):
                o = h * HEAD_DIM
                qh = qkvj[:, o:o + HEAD_DIM]
                kh = qkvj[:, D_TRANS + o:D_TRANS + o + HEAD_DIM]
                vh = qkvj[:, 2 * D_TRANS + o:2 * D_TRANS + o + HEAD_DIM]
                s = _dot_t(qh, kh) * (1.0 / math.sqrt(HEAD_DIM))
                s = s + nm
                m = jnp.max(s, axis=1, keepdims=True)
                e = jnp.exp(s - m)
                oh = _dot(e, vh) / jnp.sum(e, axis=1, keepdims=True)
                ohp = _dot(oh, Wo_h[h])
                oj = ohp if oj is None else oj + ohp
            blocks.append(oj)
        o = jnp.concatenate(blocks, axis=0) + bo
        y = _ln(y + o, g1, be1)
        ff = _dot(jax.nn.relu(_dot(y, W1) + b1), W2) + b2
        return _ln(y + ff, g2, be2)

    y = enc_layer(y, l0_Wqkv_ref[...], l0_bqkv_ref[...],
                  [l0_Wo0_ref[...], l0_Wo1_ref[...],
                   l0_Wo2_ref[...], l0_Wo3_ref[...]],
                  l0_bo_ref[...], l0_W1_ref[...], l0_b1_ref[...],
                  l0_W2_ref[...], l0_b2_ref[...], l0_g1_ref[...],
                  l0_be1_ref[...], l0_g2_ref[...], l0_be2_ref[...])
    y = enc_layer(y, l1_Wqkv_ref[...], l1_bqkv_ref[...],
                  [l1_Wo0_ref[...], l1_Wo1_ref[...],
                   l1_Wo2_ref[...], l1_Wo3_ref[...]],
                  l1_bo_ref[...], l1_W1_ref[...], l1_b1_ref[...],
                  l1_W2_ref[...], l1_b2_ref[...], l1_g1_ref[...],
                  l1_be1_ref[...], l1_g2_ref[...], l1_be2_ref[...])

    yv = y * mq_ref[...]
    agg = _dot(s_ref[...], yv)
    lenv = len_ref[...][:, 0, :]
    agg = agg / (lenv + 1.0)
    feat = agg[:, :D_FINAL]
    h1 = jax.nn.relu(_dot(feat, Wm1_ref[...]) + bm1_ref[...])
    outm = _dot(h1, Wm2_ref[...]) + bm2_ref[...]
    out_ref[...] = outm[:, None, :]


def _dist_kernel(a_ref, o_ref):
    a = a_ref[...]
    nb = a.shape[0]
    rb = 8

    def body(i, acc):
        blk = a_ref[pl.ds(i * rb, rb), :]
        diff = a[None, :, :] - blk[:, None, :]
        d2 = jnp.sum(diff * diff, axis=2)
        d = jnp.sqrt(jnp.maximum(d2, 1e-24))
        return acc + jnp.sum(d)

    tot = jax.lax.fori_loop(0, nb // rb, body, jnp.float32(0.0))
    o_ref[0, 0] = tot / float(nb * nb)


def kernel(src, static, times, lengths, adj, W_enc, b_enc, W_emb, b_emb,
           Wq, bq, Wk, bk, Wv, bv, Wskip, bskip,
           l0_Wqkv, l0_bqkv, l0_Wo, l0_bo, l0_W1, l0_b1, l0_W2, l0_b2,
           l0_ln1_g, l0_ln1_b, l0_ln2_g, l0_ln2_b,
           l1_Wqkv, l1_bqkv, l1_Wo, l1_bo, l1_W1, l1_b1, l1_W2, l1_b2,
           l1_ln1_g, l1_ln1_b, l1_ln2_g, l1_ln2_b,
           Wm1, bm1, Wm2, bm2):
    T, B = src.shape[0], src.shape[1]
    f32 = jnp.float32

    def pad_t(a):
        a = jnp.pad(a, ((0, 0), (0, T_PAD - T), (0, 0)))
        return a.reshape(B * T_PAD, a.shape[-1])

    xsrc = pad_t(jnp.transpose(src[:, :, :D_INP], (1, 0, 2)))
    t_r = pad_t(jnp.transpose(times, (1, 0))[:, :, None])
    valid = (jnp.arange(T_PAD)[None, :] < lengths[:, None]).astype(f32)
    mk = valid[:, None, :]
    mq = valid.reshape(B * T_PAD, 1)
    lenf = lengths.astype(f32)[:, None, None]
    adj_sl = adj.at[jnp.diag_indices(D_INP)].set(1.0)
    adjt = jnp.transpose(adj_sl, (1, 0))
    adjt = jnp.pad(adjt, ((0, NG - D_INP), (0, NG - D_INP)),
                   constant_values=1.0)

    r2 = lambda a: a.reshape(1, -1)

    def wo_h(Wo):
        return [Wo[h * HEAD_DIM:(h + 1) * HEAD_DIM] for h in range(NHEAD)]

    def bspec(shape):
        nd = len(shape)
        return pl.BlockSpec(shape, lambda b, _n=nd: (0,) * _n)

    in_arrays = [
        xsrc, t_r, mk, mq, lenf, adjt,
        jnp.asarray(_TS2), jnp.asarray(_PH), jnp.asarray(_P), jnp.asarray(_S),
        W_enc, r2(b_enc),
        Wq, r2(bq), Wk, r2(bk), Wv, r2(bv), Wskip, r2(bskip),
        l0_Wqkv, r2(l0_bqkv), *wo_h(l0_Wo), r2(l0_bo),
        l0_W1, r2(l0_b1), l0_W2, r2(l0_b2),
        r2(l0_ln1_g), r2(l0_ln1_b), r2(l0_ln2_g), r2(l0_ln2_b),
        l1_Wqkv, r2(l1_bqkv), *wo_h(l1_Wo), r2(l1_bo),
        l1_W1, r2(l1_b1), l1_W2, r2(l1_b2),
        r2(l1_ln1_g), r2(l1_ln1_b), r2(l1_ln2_g), r2(l1_ln2_b),
        Wm1, r2(bm1), Wm2, r2(bm2),
    ]
    in_specs = [
        pl.BlockSpec((BB * T_PAD, D_INP), lambda b: (b, 0)),
        pl.BlockSpec((BB * T_PAD, 1), lambda b: (b, 0)),
        pl.BlockSpec((BB, 1, T_PAD), lambda b: (b, 0, 0)),
        pl.BlockSpec((BB * T_PAD, 1), lambda b: (b, 0)),
        pl.BlockSpec((BB, 1, 1), lambda b: (b, 0, 0)),
    ] + [bspec(a.shape) for a in in_arrays[5:]]

    out, attn = pl.pallas_call(
        _mega_kernel,
        grid=(B // BB,),
        in_specs=in_specs,
        out_specs=[
            pl.BlockSpec((BB, 1, N_CLASSES), lambda b: (b, 0, 0)),
            pl.BlockSpec((BB, D_INP, D_INP), lambda b: (b, 0, 0)),
        ],
        out_shape=[
            jax.ShapeDtypeStruct((B, 1, N_CLASSES), f32),
            jax.ShapeDtypeStruct((B, D_INP, D_INP), f32),
        ],
    )(*in_arrays)

    a_flat = attn.reshape(B, D_INP * D_INP)
    distance = pl.pallas_call(
        _dist_kernel,
        out_specs=pl.BlockSpec(memory_space=pltpu.SMEM),
        out_shape=jax.ShapeDtypeStruct((1, 1), f32),
    )(a_flat)

    return out.reshape(B, N_CLASSES), distance.reshape(())


# speedup vs baseline: 1.2095x; 1.1946x over previous
import math

import jax
import jax.numpy as jnp
import numpy as np
from jax.experimental import pallas as pl
from jax.experimental.pallas import tpu as pltpu

D_INP = 36
D_MODEL = 144
D_PE = 36
NHEAD = 4
MAX_LEN = 215
T_PAD = 216
D_TRANS = 180
HEAD_DIM = 45
NHID = 128
D_FINAL = 108
N_CLASSES = 2
BB = 4
NG = 40

_TS = np.asarray(MAX_LEN ** np.linspace(0.0, 1.0, D_PE // 2), np.float32)
_TS2 = np.concatenate([_TS, _TS])[None, :]
_PH = np.concatenate([np.zeros(D_PE // 2, np.float32),
                      np.full(D_PE // 2, np.float32(np.pi / 2))])[None, :]
_P = np.zeros((BB * T_PAD, BB * NG), np.float32)
for _j in range(BB):
    for _r in range(D_INP):
        _P[_j * T_PAD + _r, _j * NG + _r] = 1.0
_S = np.zeros((BB, BB * T_PAD), np.float32)
for _j in range(BB):
    _S[_j, _j * T_PAD:(_j + 1) * T_PAD] = 1.0


def _dot(a, b):
    return jax.lax.dot_general(a, b, (((1,), (0,)), ((), ())),
                               preferred_element_type=jnp.float32)


def _dot_t(a, b):
    return jax.lax.dot_general(a, b, (((1,), (1,)), ((), ())),
                               preferred_element_type=jnp.float32)


def _ln(x, g, b):
    mu = jnp.mean(x, axis=1, keepdims=True)
    d = x - mu
    var = jnp.mean(d * d, axis=1, keepdims=True)
    return d / jnp.sqrt(var + 1e-5) * g + b


def _mega_kernel(xsrc_ref, t_ref, mk_ref, mq_ref, len_ref, adjt_ref,
                 ts2_ref, ph_ref, p_ref, s_ref,
                 Wenc_ref, benc_ref,
                 Wq_ref, bq_ref, Wk_ref, bk_ref, Wv_ref, bv_ref,
                 Wskip_ref, bskip_ref,
                 l0_Wqkv_ref, l0_bqkv_ref,
                 l0_Wo0_ref, l0_Wo1_ref, l0_Wo2_ref, l0_Wo3_ref, l0_bo_ref,
                 l0_W1_ref, l0_b1_ref, l0_W2_ref, l0_b2_ref,
                 l0_g1_ref, l0_be1_ref, l0_g2_ref, l0_be2_ref,
                 l1_Wqkv_ref, l1_bqkv_ref,
                 l1_Wo0_ref, l1_Wo1_ref, l1_Wo2_ref, l1_Wo3_ref, l1_bo_ref,
                 l1_W1_ref, l1_b1_ref, l1_W2_ref, l1_b2_ref,
                 l1_g1_ref, l1_be1_ref, l1_g2_ref, l1_be2_ref,
                 Wm1_ref, bm1_ref, Wm2_ref, bm2_ref,
                 out_ref, attn_ref):
    f32 = jnp.float32

    x = xsrc_ref[...]
    x = (_dot(x, Wenc_ref[...]) + benc_ref[...]) * 12.0

    xg = jnp.concatenate([x[j * T_PAD:j * T_PAD + NG] for j in range(BB)],
                         axis=0)
    q = _dot(xg, Wq_ref[...]) + bq_ref[...]
    k = _dot(xg, Wk_ref[...]) + bk_ref[...]
    v = _dot(xg, Wv_ref[...]) + bv_ref[...]
    cmask = jnp.where(
        jax.lax.broadcasted_iota(jnp.int32, (1, NG), 1) >= D_INP,
        f32(-1e9), f32(0.0))
    rmask = jnp.where(
        jax.lax.broadcasted_iota(jnp.int32, (NG, 1), 0) >= D_INP,
        f32(0.0), f32(1.0))
    ag_blocks = []
    for j in range(BB):
        qj = q[j * NG:(j + 1) * NG]
        kj = k[j * NG:(j + 1) * NG]
        vj = v[j * NG:(j + 1) * NG]
        s = _dot_t(qj, kj) * (1.0 / 12.0)
        s = s * adjt_ref[...] + cmask
        m = jnp.max(s, axis=1, keepdims=True)
        e = jnp.exp(s - m)
        attn = e / (jnp.sum(e, axis=1, keepdims=True) + 1e-16)
        attn_ref[j] = attn[:D_INP, :D_INP]
        ag_blocks.append(_dot(attn * rmask, vj))
    ag_all = jnp.concatenate(ag_blocks, axis=0)
    skip = _dot(x, Wskip_ref[...]) + bskip_ref[...]
    outs = skip + _dot(p_ref[...], ag_all)

    t = t_ref[...]
    pe = jnp.sin(t / ts2_ref[...] + ph_ref[...])

    y = jnp.concatenate([outs, pe], axis=1)

    negm = (mk_ref[...] - 1.0) * 1e9

    def enc_layer(y, Wqkv, bqkv, Wo_h, bo, W1, b1, W2, b2, g1, be1, g2, be2):
        qkv = _dot(y, Wqkv) + bqkv
        blocks = []
        for j in range(BB):
            qkvj = qkv[j * T_PAD:(j + 1) * T_PAD]
            nm = negm[j]
            oj = None
            for h in range(NHEAD):
                o = h * HEAD_DIM
                qh = qkvj[:, o:o + HEAD_DIM]
                kh = qkvj[:, D_TRANS + o:D_TRANS + o + HEAD_DIM]
                vh = qkvj[:, 2 * D_TRANS + o:2 * D_TRANS + o + HEAD_DIM]
                s = _dot_t(qh, kh) * (1.0 / math.sqrt(HEAD_DIM))
                s = s + nm
                m = jnp.max(s, axis=1, keepdims=True)
                e = jnp.exp(s - m)
                vh1 = jnp.concatenate(
                    [vh, jnp.ones((T_PAD, 1), jnp.float32)], axis=1)
                ov = _dot(e, vh1)
                oh = ov[:, :HEAD_DIM] / ov[:, HEAD_DIM:HEAD_DIM + 1]
                ohp = _dot(oh, Wo_h[h])
                oj = ohp if oj is None else oj + ohp
            blocks.append(oj)
        o = jnp.concatenate(blocks, axis=0) + bo
        y = _ln(y + o, g1, be1)
        ff = _dot(jax.nn.relu(_dot(y, W1) + b1), W2) + b2
        return _ln(y + ff, g2, be2)

    y = enc_layer(y, l0_Wqkv_ref[...], l0_bqkv_ref[...],
                  [l0_Wo0_ref[...], l0_Wo1_ref[...],
                   l0_Wo2_ref[...], l0_Wo3_ref[...]],
                  l0_bo_ref[...], l0_W1_ref[...], l0_b1_ref[...],
                  l0_W2_ref[...], l0_b2_ref[...], l0_g1_ref[...],
                  l0_be1_ref[...], l0_g2_ref[...], l0_be2_ref[...])
    y = enc_layer(y, l1_Wqkv_ref[...], l1_bqkv_ref[...],
                  [l1_Wo0_ref[...], l1_Wo1_ref[...],
                   l1_Wo2_ref[...], l1_Wo3_ref[...]],
                  l1_bo_ref[...], l1_W1_ref[...], l1_b1_ref[...],
                  l1_W2_ref[...], l1_b2_ref[...], l1_g1_ref[...],
                  l1_be1_ref[...], l1_g2_ref[...], l1_be2_ref[...])

    yv = y * mq_ref[...]
    agg = _dot(s_ref[...], yv)
    lenv = len_ref[...][:, 0, :]
    agg = agg / (lenv + 1.0)
    feat = agg[:, :D_FINAL]
    h1 = jax.nn.relu(_dot(feat, Wm1_ref[...]) + bm1_ref[...])
    outm = _dot(h1, Wm2_ref[...]) + bm2_ref[...]
    out_ref[...] = outm[:, None, :]


def _dist_kernel(a_ref, o_ref):
    a = a_ref[...]
    nb = a.shape[0]
    rb = 8

    def body(i, acc):
        blk = a_ref[pl.ds(i * rb, rb), :]
        diff = a[None, :, :] - blk[:, None, :]
        d2 = jnp.sum(diff * diff, axis=2)
        d = jnp.sqrt(jnp.maximum(d2, 1e-24))
        return acc + jnp.sum(d)

    tot = jax.lax.fori_loop(0, nb // rb, body, jnp.float32(0.0))
    o_ref[0, 0] = tot / float(nb * nb)


def kernel(src, static, times, lengths, adj, W_enc, b_enc, W_emb, b_emb,
           Wq, bq, Wk, bk, Wv, bv, Wskip, bskip,
           l0_Wqkv, l0_bqkv, l0_Wo, l0_bo, l0_W1, l0_b1, l0_W2, l0_b2,
           l0_ln1_g, l0_ln1_b, l0_ln2_g, l0_ln2_b,
           l1_Wqkv, l1_bqkv, l1_Wo, l1_bo, l1_W1, l1_b1, l1_W2, l1_b2,
           l1_ln1_g, l1_ln1_b, l1_ln2_g, l1_ln2_b,
           Wm1, bm1, Wm2, bm2):
    T, B = src.shape[0], src.shape[1]
    f32 = jnp.float32

    def pad_t(a):
        a = jnp.pad(a, ((0, 0), (0, T_PAD - T), (0, 0)))
        return a.reshape(B * T_PAD, a.shape[-1])

    xsrc = pad_t(jnp.transpose(src[:, :, :D_INP], (1, 0, 2)))
    t_r = pad_t(jnp.transpose(times, (1, 0))[:, :, None])
    valid = (jnp.arange(T_PAD)[None, :] < lengths[:, None]).astype(f32)
    mk = valid[:, None, :]
    mq = valid.reshape(B * T_PAD, 1)
    lenf = lengths.astype(f32)[:, None, None]
    adj_sl = adj.at[jnp.diag_indices(D_INP)].set(1.0)
    adjt = jnp.transpose(adj_sl, (1, 0))
    adjt = jnp.pad(adjt, ((0, NG - D_INP), (0, NG - D_INP)),
                   constant_values=1.0)

    r2 = lambda a: a.reshape(1, -1)

    def wo_h(Wo):
        return [Wo[h * HEAD_DIM:(h + 1) * HEAD_DIM] for h in range(NHEAD)]

    def bspec(shape):
        nd = len(shape)
        return pl.BlockSpec(shape, lambda b, _n=nd: (0,) * _n)

    in_arrays = [
        xsrc, t_r, mk, mq, lenf, adjt,
        jnp.asarray(_TS2), jnp.asarray(_PH), jnp.asarray(_P), jnp.asarray(_S),
        W_enc, r2(b_enc),
        Wq, r2(bq), Wk, r2(bk), Wv, r2(bv), Wskip, r2(bskip),
        l0_Wqkv, r2(l0_bqkv), *wo_h(l0_Wo), r2(l0_bo),
        l0_W1, r2(l0_b1), l0_W2, r2(l0_b2),
        r2(l0_ln1_g), r2(l0_ln1_b), r2(l0_ln2_g), r2(l0_ln2_b),
        l1_Wqkv, r2(l1_bqkv), *wo_h(l1_Wo), r2(l1_bo),
        l1_W1, r2(l1_b1), l1_W2, r2(l1_b2),
        r2(l1_ln1_g), r2(l1_ln1_b), r2(l1_ln2_g), r2(l1_ln2_b),
        Wm1, r2(bm1), Wm2, r2(bm2),
    ]
    in_specs = [
        pl.BlockSpec((BB * T_PAD, D_INP), lambda b: (b, 0)),
        pl.BlockSpec((BB * T_PAD, 1), lambda b: (b, 0)),
        pl.BlockSpec((BB, 1, T_PAD), lambda b: (b, 0, 0)),
        pl.BlockSpec((BB * T_PAD, 1), lambda b: (b, 0)),
        pl.BlockSpec((BB, 1, 1), lambda b: (b, 0, 0)),
    ] + [bspec(a.shape) for a in in_arrays[5:]]

    out, attn = pl.pallas_call(
        _mega_kernel,
        grid=(B // BB,),
        in_specs=in_specs,
        out_specs=[
            pl.BlockSpec((BB, 1, N_CLASSES), lambda b: (b, 0, 0)),
            pl.BlockSpec((BB, D_INP, D_INP), lambda b: (b, 0, 0)),
        ],
        out_shape=[
            jax.ShapeDtypeStruct((B, 1, N_CLASSES), f32),
            jax.ShapeDtypeStruct((B, D_INP, D_INP), f32),
        ],
    )(*in_arrays)

    a_flat = attn.reshape(B, D_INP * D_INP)
    distance = pl.pallas_call(
        _dist_kernel,
        out_specs=pl.BlockSpec(memory_space=pltpu.SMEM),
        out_shape=jax.ShapeDtypeStruct((1, 1), f32),
    )(a_flat)

    return out.reshape(B, N_CLASSES), distance.reshape(())


# speedup vs baseline: 1.2845x; 1.0621x over previous
import math

import jax
import jax.numpy as jnp
import numpy as np
from jax.experimental import pallas as pl
from jax.experimental.pallas import tpu as pltpu

D_INP = 36
D_MODEL = 144
D_PE = 36
NHEAD = 4
MAX_LEN = 215
T_PAD = 216
D_TRANS = 180
HEAD_DIM = 45
NHID = 128
D_FINAL = 108
N_CLASSES = 2
BB = 4
NG = 40

_TS = np.asarray(MAX_LEN ** np.linspace(0.0, 1.0, D_PE // 2), np.float32)
_TS2 = np.concatenate([_TS, _TS])[None, :]
_PH = np.concatenate([np.zeros(D_PE // 2, np.float32),
                      np.full(D_PE // 2, np.float32(np.pi / 2))])[None, :]
_P = np.zeros((BB * T_PAD, BB * NG), np.float32)
for _j in range(BB):
    for _r in range(D_INP):
        _P[_j * T_PAD + _r, _j * NG + _r] = 1.0
_S = np.zeros((BB, BB * T_PAD), np.float32)
for _j in range(BB):
    _S[_j, _j * T_PAD:(_j + 1) * T_PAD] = 1.0


def _dot(a, b):
    return jax.lax.dot_general(a, b, (((1,), (0,)), ((), ())),
                               preferred_element_type=jnp.float32)


def _dot_t(a, b):
    return jax.lax.dot_general(a, b, (((1,), (1,)), ((), ())),
                               preferred_element_type=jnp.float32)


def _ln(x, g, b):
    mu = jnp.mean(x, axis=1, keepdims=True)
    d = x - mu
    var = jnp.mean(d * d, axis=1, keepdims=True)
    return d / jnp.sqrt(var + 1e-5) * g + b


def _mega_kernel(xsrc_ref, t_ref, mk_ref, mq_ref, len_ref, adjt_ref,
                 ts2_ref, ph_ref, p_ref, s_ref,
                 Wenc_ref, benc_ref,
                 Wq_ref, bq_ref, Wk_ref, bk_ref, Wv_ref, bv_ref,
                 Wskip_ref, bskip_ref,
                 l0_Wqkv_ref, l0_bqkv_ref,
                 l0_Wo0_ref, l0_Wo1_ref, l0_Wo2_ref, l0_Wo3_ref, l0_bo_ref,
                 l0_W1_ref, l0_b1_ref, l0_W2_ref, l0_b2_ref,
                 l0_g1_ref, l0_be1_ref, l0_g2_ref, l0_be2_ref,
                 l1_Wqkv_ref, l1_bqkv_ref,
                 l1_Wo0_ref, l1_Wo1_ref, l1_Wo2_ref, l1_Wo3_ref, l1_bo_ref,
                 l1_W1_ref, l1_b1_ref, l1_W2_ref, l1_b2_ref,
                 l1_g1_ref, l1_be1_ref, l1_g2_ref, l1_be2_ref,
                 Wm1_ref, bm1_ref, Wm2_ref, bm2_ref,
                 out_ref, attn_ref):
    f32 = jnp.float32

    x = xsrc_ref[...]
    x = (_dot(x, Wenc_ref[...]) + benc_ref[...]) * 12.0

    xg = jnp.concatenate([x[j * T_PAD:j * T_PAD + NG] for j in range(BB)],
                         axis=0)
    q = _dot(xg, Wq_ref[...]) + bq_ref[...]
    k = _dot(xg, Wk_ref[...]) + bk_ref[...]
    v = _dot(xg, Wv_ref[...]) + bv_ref[...]
    cmask = jnp.where(
        jax.lax.broadcasted_iota(jnp.int32, (1, NG), 1) >= D_INP,
        f32(-1e9), f32(0.0))
    rmask = jnp.where(
        jax.lax.broadcasted_iota(jnp.int32, (NG, 1), 0) >= D_INP,
        f32(0.0), f32(1.0))
    ag_blocks = []
    for j in range(BB):
        qj = q[j * NG:(j + 1) * NG]
        kj = k[j * NG:(j + 1) * NG]
        vj = v[j * NG:(j + 1) * NG]
        s = _dot_t(qj, kj) * (1.0 / 12.0)
        s = s * adjt_ref[...] + cmask
        m = jnp.max(s, axis=1, keepdims=True)
        e = jnp.exp(s - m)
        attn = e / (jnp.sum(e, axis=1, keepdims=True) + 1e-16)
        attn_ref[j] = attn[:D_INP, :D_INP]
        ag_blocks.append(_dot(attn * rmask, vj))
    ag_all = jnp.concatenate(ag_blocks, axis=0)
    skip = _dot(x, Wskip_ref[...]) + bskip_ref[...]
    outs = skip + _dot(p_ref[...], ag_all)

    t = t_ref[...]
    xarg = t / ts2_ref[...] + ph_ref[...]
    kq = jnp.round(xarg * f32(1.0 / (2.0 * np.pi)))
    r = xarg - kq * f32(6.28125) - kq * f32(1.9353071795864769e-3)
    r2 = r * r
    pe = r * (1.0 + r2 * (f32(-1.0 / 6.0) + r2 * (
        f32(1.0 / 120.0) + r2 * (f32(-1.0 / 5040.0) + r2 * (
            f32(1.0 / 362880.0) + r2 * f32(-1.0 / 39916800.0))))))

    y = jnp.concatenate([outs, pe], axis=1)

    negm = (mk_ref[...] - 1.0) * 1e9

    def enc_layer(y, Wqkv, bqkv, Wo_h, bo, W1, b1, W2, b2, g1, be1, g2, be2):
        qkv = _dot(y, Wqkv) + bqkv
        blocks = []
        for j in range(BB):
            qkvj = qkv[j * T_PAD:(j + 1) * T_PAD]
            nm = negm[j]
            oj = None
            for h in range(NHEAD):
                o = h * HEAD_DIM
                qh = qkvj[:, o:o + HEAD_DIM]
                kh = qkvj[:, D_TRANS + o:D_TRANS + o + HEAD_DIM]
                vh = qkvj[:, 2 * D_TRANS + o:2 * D_TRANS + o + HEAD_DIM]
                s = _dot_t(qh, kh) * (1.0 / math.sqrt(HEAD_DIM))
                s = s + nm
                m = jnp.max(s, axis=1, keepdims=True)
                e = jnp.exp(s - m)
                vh1 = jnp.concatenate(
                    [vh, jnp.ones((T_PAD, 1), jnp.float32)], axis=1)
                ov = _dot(e, vh1)
                oh = ov[:, :HEAD_DIM] / ov[:, HEAD_DIM:HEAD_DIM + 1]
                ohp = _dot(oh, Wo_h[h])
                oj = ohp if oj is None else oj + ohp
            blocks.append(oj)
        o = jnp.concatenate(blocks, axis=0) + bo
        y = _ln(y + o, g1, be1)
        ff = _dot(jax.nn.relu(_dot(y, W1) + b1), W2) + b2
        return _ln(y + ff, g2, be2)

    y = enc_layer(y, l0_Wqkv_ref[...], l0_bqkv_ref[...],
                  [l0_Wo0_ref[...], l0_Wo1_ref[...],
                   l0_Wo2_ref[...], l0_Wo3_ref[...]],
                  l0_bo_ref[...], l0_W1_ref[...], l0_b1_ref[...],
                  l0_W2_ref[...], l0_b2_ref[...], l0_g1_ref[...],
                  l0_be1_ref[...], l0_g2_ref[...], l0_be2_ref[...])
    y = enc_layer(y, l1_Wqkv_ref[...], l1_bqkv_ref[...],
                  [l1_Wo0_ref[...], l1_Wo1_ref[...],
                   l1_Wo2_ref[...], l1_Wo3_ref[...]],
                  l1_bo_ref[...], l1_W1_ref[...], l1_b1_ref[...],
                  l1_W2_ref[...], l1_b2_ref[...], l1_g1_ref[...],
                  l1_be1_ref[...], l1_g2_ref[...], l1_be2_ref[...])

    yv = y * mq_ref[...]
    agg = _dot(s_ref[...], yv)
    lenv = len_ref[...][:, 0, :]
    agg = agg / (lenv + 1.0)
    feat = agg[:, :D_FINAL]
    h1 = jax.nn.relu(_dot(feat, Wm1_ref[...]) + bm1_ref[...])
    outm = _dot(h1, Wm2_ref[...]) + bm2_ref[...]
    out_ref[...] = outm[:, None, :]


def _dist_kernel(a_ref, o_ref):
    a = a_ref[...]
    nb = a.shape[0]
    rb = 8

    def body(i, acc):
        blk = a_ref[pl.ds(i * rb, rb), :]
        diff = a[None, :, :] - blk[:, None, :]
        d2 = jnp.sum(diff * diff, axis=2)
        d = jnp.sqrt(jnp.maximum(d2, 1e-24))
        return acc + jnp.sum(d)

    tot = jax.lax.fori_loop(0, nb // rb, body, jnp.float32(0.0))
    o_ref[0, 0] = tot / float(nb * nb)


def kernel(src, static, times, lengths, adj, W_enc, b_enc, W_emb, b_emb,
           Wq, bq, Wk, bk, Wv, bv, Wskip, bskip,
           l0_Wqkv, l0_bqkv, l0_Wo, l0_bo, l0_W1, l0_b1, l0_W2, l0_b2,
           l0_ln1_g, l0_ln1_b, l0_ln2_g, l0_ln2_b,
           l1_Wqkv, l1_bqkv, l1_Wo, l1_bo, l1_W1, l1_b1, l1_W2, l1_b2,
           l1_ln1_g, l1_ln1_b, l1_ln2_g, l1_ln2_b,
           Wm1, bm1, Wm2, bm2):
    T, B = src.shape[0], src.shape[1]
    f32 = jnp.float32

    def pad_t(a):
        a = jnp.pad(a, ((0, 0), (0, T_PAD - T), (0, 0)))
        return a.reshape(B * T_PAD, a.shape[-1])

    xsrc = pad_t(jnp.transpose(src[:, :, :D_INP], (1, 0, 2)))
    t_r = pad_t(jnp.transpose(times, (1, 0))[:, :, None])
    valid = (jnp.arange(T_PAD)[None, :] < lengths[:, None]).astype(f32)
    mk = valid[:, None, :]
    mq = valid.reshape(B * T_PAD, 1)
    lenf = lengths.astype(f32)[:, None, None]
    adj_sl = adj.at[jnp.diag_indices(D_INP)].set(1.0)
    adjt = jnp.transpose(adj_sl, (1, 0))
    adjt = jnp.pad(adjt, ((0, NG - D_INP), (0, NG - D_INP)),
                   constant_values=1.0)

    r2 = lambda a: a.reshape(1, -1)

    def wo_h(Wo):
        return [Wo[h * HEAD_DIM:(h + 1) * HEAD_DIM] for h in range(NHEAD)]

    def qkv_ext(Wqkv, bqkv):
        Wv_ = Wqkv[:, 2 * D_TRANS:]
        bv_ = bqkv[2 * D_TRANS:]
        wcols = [Wqkv[:, :2 * D_TRANS]]
        bcols = [bqkv[:2 * D_TRANS]]
        for h in range(NHEAD):
            wcols += [Wv_[:, h * HEAD_DIM:(h + 1) * HEAD_DIM],
                      jnp.zeros((D_TRANS, 1), f32)]
            bcols += [bv_[h * HEAD_DIM:(h + 1) * HEAD_DIM],
                      jnp.ones((1,), f32)]
        return jnp.concatenate(wcols, axis=1), jnp.concatenate(bcols)[None, :]

    l0_Wqkv_e, l0_bqkv_e = qkv_ext(l0_Wqkv, l0_bqkv)
    l1_Wqkv_e, l1_bqkv_e = qkv_ext(l1_Wqkv, l1_bqkv)

    def bspec(shape):
        nd = len(shape)
        return pl.BlockSpec(shape, lambda b, _n=nd: (0,) * _n)

    in_arrays = [
        xsrc, t_r, mk, mq, lenf, adjt,
        jnp.asarray(_TS2), jnp.asarray(_PH), jnp.asarray(_P), jnp.asarray(_S),
        W_enc, r2(b_enc),
        Wq, r2(bq), Wk, r2(bk), Wv, r2(bv), Wskip, r2(bskip),
        l0_Wqkv, r2(l0_bqkv), *wo_h(l0_Wo), r2(l0_bo),
        l0_W1, r2(l0_b1), l0_W2, r2(l0_b2),
        r2(l0_ln1_g), r2(l0_ln1_b), r2(l0_ln2_g), r2(l0_ln2_b),
        l1_Wqkv, r2(l1_bqkv), *wo_h(l1_Wo), r2(l1_bo),
        l1_W1, r2(l1_b1), l1_W2, r2(l1_b2),
        r2(l1_ln1_g), r2(l1_ln1_b), r2(l1_ln2_g), r2(l1_ln2_b),
        Wm1, r2(bm1), Wm2, r2(bm2),
    ]
    in_specs = [
        pl.BlockSpec((BB * T_PAD, D_INP), lambda b: (b, 0)),
        pl.BlockSpec((BB * T_PAD, 1), lambda b: (b, 0)),
        pl.BlockSpec((BB, 1, T_PAD), lambda b: (b, 0, 0)),
        pl.BlockSpec((BB * T_PAD, 1), lambda b: (b, 0)),
        pl.BlockSpec((BB, 1, 1), lambda b: (b, 0, 0)),
    ] + [bspec(a.shape) for a in in_arrays[5:]]

    out, attn = pl.pallas_call(
        _mega_kernel,
        grid=(B // BB,),
        in_specs=in_specs,
        out_specs=[
            pl.BlockSpec((BB, 1, N_CLASSES), lambda b: (b, 0, 0)),
            pl.BlockSpec((BB, D_INP, D_INP), lambda b: (b, 0, 0)),
        ],
        out_shape=[
            jax.ShapeDtypeStruct((B, 1, N_CLASSES), f32),
            jax.ShapeDtypeStruct((B, D_INP, D_INP), f32),
        ],
    )(*in_arrays)

    a_flat = attn.reshape(B, D_INP * D_INP)
    distance = pl.pallas_call(
        _dist_kernel,
        out_specs=pl.BlockSpec(memory_space=pltpu.SMEM),
        out_shape=jax.ShapeDtypeStruct((1, 1), f32),
    )(a_flat)

    return out.reshape(B, N_CLASSES), distance.reshape(())


# speedup vs baseline: 1.3232x; 1.0301x over previous
import math

import jax
import jax.numpy as jnp
import numpy as np
from jax.experimental import pallas as pl
from jax.experimental.pallas import tpu as pltpu

D_INP = 36
D_MODEL = 144
D_PE = 36
NHEAD = 4
MAX_LEN = 215
T_PAD = 216
D_TRANS = 180
HEAD_DIM = 45
NHID = 128
D_FINAL = 108
N_CLASSES = 2
BB = 4
NG = 40

_TS = np.asarray(MAX_LEN ** np.linspace(0.0, 1.0, D_PE // 2), np.float32)
_TS2 = np.concatenate([_TS, _TS])[None, :]
_PH = np.concatenate([np.zeros(D_PE // 2, np.float32),
                      np.full(D_PE // 2, np.float32(np.pi / 2))])[None, :]
_P = np.zeros((BB * T_PAD, BB * NG), np.float32)
for _j in range(BB):
    for _r in range(D_INP):
        _P[_j * T_PAD + _r, _j * NG + _r] = 1.0
_S = np.zeros((BB, BB * T_PAD), np.float32)
for _j in range(BB):
    _S[_j, _j * T_PAD:(_j + 1) * T_PAD] = 1.0


def _dot(a, b):
    return jax.lax.dot_general(a, b, (((1,), (0,)), ((), ())),
                               preferred_element_type=jnp.float32)


def _dot_t(a, b):
    return jax.lax.dot_general(a, b, (((1,), (1,)), ((), ())),
                               preferred_element_type=jnp.float32)


def _ln(x, g, b):
    mu = jnp.mean(x, axis=1, keepdims=True)
    d = x - mu
    var = jnp.mean(d * d, axis=1, keepdims=True)
    return d * jax.lax.rsqrt(var + 1e-5) * g + b


def _mega_kernel(xsrc_ref, t_ref, mk_ref, mq_ref, len_ref, adjt_ref,
                 ts2_ref, ph_ref, p_ref, s_ref,
                 Wenc_ref, benc_ref,
                 Wq_ref, bq_ref, Wk_ref, bk_ref, Wv_ref, bv_ref,
                 Wskip_ref, bskip_ref,
                 l0_Wqkv_ref, l0_bqkv_ref,
                 l0_Wo0_ref, l0_Wo1_ref, l0_Wo2_ref, l0_Wo3_ref, l0_bo_ref,
                 l0_W1_ref, l0_b1_ref, l0_W2_ref, l0_b2_ref,
                 l0_g1_ref, l0_be1_ref, l0_g2_ref, l0_be2_ref,
                 l1_Wqkv_ref, l1_bqkv_ref,
                 l1_Wo0_ref, l1_Wo1_ref, l1_Wo2_ref, l1_Wo3_ref, l1_bo_ref,
                 l1_W1_ref, l1_b1_ref, l1_W2_ref, l1_b2_ref,
                 l1_g1_ref, l1_be1_ref, l1_g2_ref, l1_be2_ref,
                 Wm1_ref, bm1_ref, Wm2_ref, bm2_ref,
                 out_ref, attn_ref):
    f32 = jnp.float32

    x = xsrc_ref[...]
    x = (_dot(x, Wenc_ref[...]) + benc_ref[...]) * 12.0

    xg = jnp.concatenate([x[j * T_PAD:j * T_PAD + NG] for j in range(BB)],
                         axis=0)
    q = _dot(xg, Wq_ref[...]) + bq_ref[...]
    k = _dot(xg, Wk_ref[...]) + bk_ref[...]
    v = _dot(xg, Wv_ref[...]) + bv_ref[...]
    cmask = jnp.where(
        jax.lax.broadcasted_iota(jnp.int32, (1, NG), 1) >= D_INP,
        f32(-1e9), f32(0.0))
    rmask = jnp.where(
        jax.lax.broadcasted_iota(jnp.int32, (NG, 1), 0) >= D_INP,
        f32(0.0), f32(1.0))
    ag_blocks = []
    for j in range(BB):
        qj = q[j * NG:(j + 1) * NG]
        kj = k[j * NG:(j + 1) * NG]
        vj = v[j * NG:(j + 1) * NG]
        s = _dot_t(qj, kj) * (1.0 / 12.0)
        s = s * adjt_ref[...] + cmask
        m = jnp.max(s, axis=1, keepdims=True)
        e = jnp.exp(s - m)
        attn = e / (jnp.sum(e, axis=1, keepdims=True) + 1e-16)
        attn_ref[j] = attn[:D_INP, :D_INP]
        ag_blocks.append(_dot(attn * rmask, vj))
    ag_all = jnp.concatenate(ag_blocks, axis=0)
    skip = _dot(x, Wskip_ref[...]) + bskip_ref[...]
    outs = skip + _dot(p_ref[...], ag_all)

    t = t_ref[...]
    xarg = t / ts2_ref[...] + ph_ref[...]
    kq = jnp.round(xarg * f32(1.0 / (2.0 * np.pi)))
    r = xarg - kq * f32(6.28125) - kq * f32(1.9353071795864769e-3)
    r2 = r * r
    pe = r * (1.0 + r2 * (f32(-1.0 / 6.0) + r2 * (
        f32(1.0 / 120.0) + r2 * (f32(-1.0 / 5040.0) + r2 * (
            f32(1.0 / 362880.0) + r2 * f32(-1.0 / 39916800.0))))))

    y = jnp.concatenate([outs, pe], axis=1)

    negm = (mk_ref[...] - 1.0) * 1e9

    def enc_layer(y, Wqkv, bqkv, Wo_h, bo, W1, b1, W2, b2, g1, be1, g2, be2):
        qkv = _dot(y, Wqkv) + bqkv
        blocks = []
        for j in range(BB):
            qkvj = qkv[j * T_PAD:(j + 1) * T_PAD]
            nm = negm[j]
            oj = None
            for h in range(NHEAD):
                o = h * HEAD_DIM
                qh = qkvj[:, o:o + HEAD_DIM]
                kh = qkvj[:, D_TRANS + o:D_TRANS + o + HEAD_DIM]
                vh = qkvj[:, 2 * D_TRANS + o:2 * D_TRANS + o + HEAD_DIM]
                s = _dot_t(qh, kh) * (1.0 / math.sqrt(HEAD_DIM))
                s = s + nm
                m = jnp.max(s, axis=1, keepdims=True)
                e = jnp.exp(s - m)
                vh1 = jnp.concatenate(
                    [vh, jnp.ones((T_PAD, 1), jnp.float32)], axis=1)
                ov = _dot(e, vh1)
                oh = ov[:, :HEAD_DIM] * (1.0 / ov[:, HEAD_DIM:HEAD_DIM + 1])
                ohp = _dot(oh, Wo_h[h])
                oj = ohp if oj is None else oj + ohp
            blocks.append(oj)
        o = jnp.concatenate(blocks, axis=0) + bo
        y = _ln(y + o, g1, be1)
        ff = _dot(jax.nn.relu(_dot(y, W1) + b1), W2) + b2
        return _ln(y + ff, g2, be2)

    y = enc_layer(y, l0_Wqkv_ref[...], l0_bqkv_ref[...],
                  [l0_Wo0_ref[...], l0_Wo1_ref[...],
                   l0_Wo2_ref[...], l0_Wo3_ref[...]],
                  l0_bo_ref[...], l0_W1_ref[...], l0_b1_ref[...],
                  l0_W2_ref[...], l0_b2_ref[...], l0_g1_ref[...],
                  l0_be1_ref[...], l0_g2_ref[...], l0_be2_ref[...])
    y = enc_layer(y, l1_Wqkv_ref[...], l1_bqkv_ref[...],
                  [l1_Wo0_ref[...], l1_Wo1_ref[...],
                   l1_Wo2_ref[...], l1_Wo3_ref[...]],
                  l1_bo_ref[...], l1_W1_ref[...], l1_b1_ref[...],
                  l1_W2_ref[...], l1_b2_ref[...], l1_g1_ref[...],
                  l1_be1_ref[...], l1_g2_ref[...], l1_be2_ref[...])

    yv = y * mq_ref[...]
    agg = _dot(s_ref[...], yv)
    lenv = len_ref[...][:, 0, :]
    agg = agg / (lenv + 1.0)
    feat = agg[:, :D_FINAL]
    h1 = jax.nn.relu(_dot(feat, Wm1_ref[...]) + bm1_ref[...])
    outm = _dot(h1, Wm2_ref[...]) + bm2_ref[...]
    out_ref[...] = outm[:, None, :]


def _dist_kernel(a_ref, o_ref):
    a = a_ref[...]
    nb = a.shape[0]
    rb = 8

    def body(i, acc):
        blk = a_ref[pl.ds(i * rb, rb), :]
        diff = a[None, :, :] - blk[:, None, :]
        d2 = jnp.sum(diff * diff, axis=2)
        d = jnp.sqrt(jnp.maximum(d2, 1e-24))
        return acc + jnp.sum(d)

    tot = jax.lax.fori_loop(0, nb // rb, body, jnp.float32(0.0))
    o_ref[0, 0] = tot / float(nb * nb)


def kernel(src, static, times, lengths, adj, W_enc, b_enc, W_emb, b_emb,
           Wq, bq, Wk, bk, Wv, bv, Wskip, bskip,
           l0_Wqkv, l0_bqkv, l0_Wo, l0_bo, l0_W1, l0_b1, l0_W2, l0_b2,
           l0_ln1_g, l0_ln1_b, l0_ln2_g, l0_ln2_b,
           l1_Wqkv, l1_bqkv, l1_Wo, l1_bo, l1_W1, l1_b1, l1_W2, l1_b2,
           l1_ln1_g, l1_ln1_b, l1_ln2_g, l1_ln2_b,
           Wm1, bm1, Wm2, bm2):
    T, B = src.shape[0], src.shape[1]
    f32 = jnp.float32

    def pad_t(a):
        a = jnp.pad(a, ((0, 0), (0, T_PAD - T), (0, 0)))
        return a.reshape(B * T_PAD, a.shape[-1])

    xsrc = pad_t(jnp.transpose(src[:, :, :D_INP], (1, 0, 2)))
    t_r = pad_t(jnp.transpose(times, (1, 0))[:, :, None])
    valid = (jnp.arange(T_PAD)[None, :] < lengths[:, None]).astype(f32)
    mk = valid[:, None, :]
    mq = valid.reshape(B * T_PAD, 1)
    lenf = lengths.astype(f32)[:, None, None]
    adj_sl = adj.at[jnp.diag_indices(D_INP)].set(1.0)
    adjt = jnp.transpose(adj_sl, (1, 0))
    adjt = jnp.pad(adjt, ((0, NG - D_INP), (0, NG - D_INP)),
                   constant_values=1.0)

    r2 = lambda a: a.reshape(1, -1)

    def wo_h(Wo):
        return [Wo[h * HEAD_DIM:(h + 1) * HEAD_DIM] for h in range(NHEAD)]

    def qkv_ext(Wqkv, bqkv):
        Wv_ = Wqkv[:, 2 * D_TRANS:]
        bv_ = bqkv[2 * D_TRANS:]
        wcols = [Wqkv[:, :2 * D_TRANS]]
        bcols = [bqkv[:2 * D_TRANS]]
        for h in range(NHEAD):
            wcols += [Wv_[:, h * HEAD_DIM:(h + 1) * HEAD_DIM],
                      jnp.zeros((D_TRANS, 1), f32)]
            bcols += [bv_[h * HEAD_DIM:(h + 1) * HEAD_DIM],
                      jnp.ones((1,), f32)]
        return jnp.concatenate(wcols, axis=1), jnp.concatenate(bcols)[None, :]

    l0_Wqkv_e, l0_bqkv_e = qkv_ext(l0_Wqkv, l0_bqkv)
    l1_Wqkv_e, l1_bqkv_e = qkv_ext(l1_Wqkv, l1_bqkv)

    def bspec(shape):
        nd = len(shape)
        return pl.BlockSpec(shape, lambda b, _n=nd: (0,) * _n)

    in_arrays = [
        xsrc, t_r, mk, mq, lenf, adjt,
        jnp.asarray(_TS2), jnp.asarray(_PH), jnp.asarray(_P), jnp.asarray(_S),
        W_enc, r2(b_enc),
        Wq, r2(bq), Wk, r2(bk), Wv, r2(bv), Wskip, r2(bskip),
        l0_Wqkv, r2(l0_bqkv), *wo_h(l0_Wo), r2(l0_bo),
        l0_W1, r2(l0_b1), l0_W2, r2(l0_b2),
        r2(l0_ln1_g), r2(l0_ln1_b), r2(l0_ln2_g), r2(l0_ln2_b),
        l1_Wqkv, r2(l1_bqkv), *wo_h(l1_Wo), r2(l1_bo),
        l1_W1, r2(l1_b1), l1_W2, r2(l1_b2),
        r2(l1_ln1_g), r2(l1_ln1_b), r2(l1_ln2_g), r2(l1_ln2_b),
        Wm1, r2(bm1), Wm2, r2(bm2),
    ]
    in_specs = [
        pl.BlockSpec((BB * T_PAD, D_INP), lambda b: (b, 0)),
        pl.BlockSpec((BB * T_PAD, 1), lambda b: (b, 0)),
        pl.BlockSpec((BB, 1, T_PAD), lambda b: (b, 0, 0)),
        pl.BlockSpec((BB * T_PAD, 1), lambda b: (b, 0)),
        pl.BlockSpec((BB, 1, 1), lambda b: (b, 0, 0)),
    ] + [bspec(a.shape) for a in in_arrays[5:]]

    out, attn = pl.pallas_call(
        _mega_kernel,
        grid=(B // BB,),
        in_specs=in_specs,
        out_specs=[
            pl.BlockSpec((BB, 1, N_CLASSES), lambda b: (b, 0, 0)),
            pl.BlockSpec((BB, D_INP, D_INP), lambda b: (b, 0, 0)),
        ],
        out_shape=[
            jax.ShapeDtypeStruct((B, 1, N_CLASSES), f32),
            jax.ShapeDtypeStruct((B, D_INP, D_INP), f32),
        ],
    )(*in_arrays)

    a_flat = attn.reshape(B, D_INP * D_INP)
    distance = pl.pallas_call(
        _dist_kernel,
        out_specs=pl.BlockSpec(memory_space=pltpu.SMEM),
        out_shape=jax.ShapeDtypeStruct((1, 1), f32),
    )(a_flat)

    return out.reshape(B, N_CLASSES), distance.reshape(())
